# baseline (speedup 1.0000x reference)
import jax
import jax.numpy as jnp
from jax.experimental import pallas as pl
from jax.experimental.pallas import tpu as pltpu

_GRID = 80


def _copy_body(n_ref, ei_ref, e_ref, u_ref, b_ref,
               no_ref, eio_ref, eo_ref, uo_ref, bo_ref,
               s0, s1, s2):
    i = pl.program_id(0)

    @pl.when(i == 0)
    def _start():
        pltpu.make_async_copy(ei_ref, eio_ref, s0).start()
        pltpu.make_async_copy(u_ref, uo_ref, s1).start()
        pltpu.make_async_copy(b_ref, bo_ref, s2).start()

    no_ref[...] = n_ref[...]
    eo_ref[...] = e_ref[...]

    @pl.when(i == pl.num_programs(0) - 1)
    def _finish():
        pltpu.make_async_copy(ei_ref, eio_ref, s0).wait()
        pltpu.make_async_copy(u_ref, uo_ref, s1).wait()
        pltpu.make_async_copy(b_ref, bo_ref, s2).wait()


def kernel(nodes, edge_index, edges=None, u=None, batch=None):
    if batch is None:
        batch = jnp.zeros((nodes.shape[0],), dtype=jnp.int32)

    n_rows, d_feat = nodes.shape
    n_edges, d_edge = edges.shape
    g = _GRID
    nb = n_rows // 10
    eb = n_edges // g

    any_spec = pl.BlockSpec(memory_space=pl.ANY)
    specs = [
        pl.BlockSpec((nb, d_feat), lambda i: (i // 8, 0)),
        any_spec,
        pl.BlockSpec((eb, d_edge), lambda i: (i, 0)),
        any_spec,
        any_spec,
    ]
    out = pl.pallas_call(
        _copy_body,
        grid=(g,),
        in_specs=specs,
        out_specs=specs,
        out_shape=[
            jax.ShapeDtypeStruct(nodes.shape, nodes.dtype),
            jax.ShapeDtypeStruct(edge_index.shape, edge_index.dtype),
            jax.ShapeDtypeStruct(edges.shape, edges.dtype),
            jax.ShapeDtypeStruct(u.shape, u.dtype),
            jax.ShapeDtypeStruct(batch.shape, batch.dtype),
        ],
        scratch_shapes=[pltpu.SemaphoreType.DMA] * 3,
    )(nodes, edge_index, edges, u, batch)

    return tuple(out)


# speedup vs baseline: 1.0244x; 1.0244x over previous
import jax
import jax.numpy as jnp
from jax.experimental import pallas as pl
from jax.experimental.pallas import tpu as pltpu

_GRID = 40


def _copy_body(n_ref, ei_ref, e_ref, u_ref, b_ref,
               no_ref, eio_ref, eo_ref, uo_ref, bo_ref,
               s0, s1, s2):
    i = pl.program_id(0)

    @pl.when(i == 0)
    def _start():
        pltpu.make_async_copy(ei_ref, eio_ref, s0).start()
        pltpu.make_async_copy(u_ref, uo_ref, s1).start()
        pltpu.make_async_copy(b_ref, bo_ref, s2).start()

    no_ref[...] = n_ref[...]
    eo_ref[...] = e_ref[...]

    @pl.when(i == pl.num_programs(0) - 1)
    def _finish():
        pltpu.make_async_copy(ei_ref, eio_ref, s0).wait()
        pltpu.make_async_copy(u_ref, uo_ref, s1).wait()
        pltpu.make_async_copy(b_ref, bo_ref, s2).wait()


def kernel(nodes, edge_index, edges=None, u=None, batch=None):
    if batch is None:
        batch = jnp.zeros((nodes.shape[0],), dtype=jnp.int32)

    n_rows, d_feat = nodes.shape
    n_edges, d_edge = edges.shape
    g = _GRID
    nb = n_rows // 10
    eb = n_edges // g

    any_spec = pl.BlockSpec(memory_space=pl.ANY)
    specs = [
        pl.BlockSpec((nb, d_feat), lambda i: (i // 4, 0)),
        any_spec,
        pl.BlockSpec((eb, d_edge), lambda i: (i, 0)),
        any_spec,
        any_spec,
    ]
    out = pl.pallas_call(
        _copy_body,
        grid=(g,),
        in_specs=specs,
        out_specs=specs,
        out_shape=[
            jax.ShapeDtypeStruct(nodes.shape, nodes.dtype),
            jax.ShapeDtypeStruct(edge_index.shape, edge_index.dtype),
            jax.ShapeDtypeStruct(edges.shape, edges.dtype),
            jax.ShapeDtypeStruct(u.shape, u.dtype),
            jax.ShapeDtypeStruct(batch.shape, batch.dtype),
        ],
        scratch_shapes=[pltpu.SemaphoreType.DMA] * 3,
    )(nodes, edge_index, edges, u, batch)

    return tuple(out)


# speedup vs baseline: 1.1459x; 1.1185x over previous
import jax
import jax.numpy as jnp
from jax.experimental import pallas as pl
from jax.experimental.pallas import tpu as pltpu

_GRID = 40


def _copy_body(n_ref, ei_ref, e_ref, u_ref, b_ref,
               no_ref, eio_ref, eo_ref, uo_ref, bo_ref,
               s0, s1, s2):
    i = pl.program_id(0)

    @pl.when(i == 0)
    def _start():
        pltpu.make_async_copy(ei_ref, eio_ref, s0).start()
        pltpu.make_async_copy(u_ref, uo_ref, s1).start()
        pltpu.make_async_copy(b_ref, bo_ref, s2).start()

    no_ref[...] = n_ref[...]

    @pl.when(i == pl.num_programs(0) - 1)
    def _finish():
        pltpu.make_async_copy(ei_ref, eio_ref, s0).wait()
        pltpu.make_async_copy(u_ref, uo_ref, s1).wait()
        pltpu.make_async_copy(b_ref, bo_ref, s2).wait()


def kernel(nodes, edge_index, edges=None, u=None, batch=None):
    if batch is None:
        batch = jnp.zeros((nodes.shape[0],), dtype=jnp.int32)

    n_rows, d_feat = nodes.shape
    n_edges, d_edge = edges.shape
    g = _GRID
    nb = n_rows // 10
    eb = n_edges // g

    any_spec = pl.BlockSpec(memory_space=pl.ANY)
    specs = [
        pl.BlockSpec((nb, d_feat), lambda i: (i // 4, 0)),
        any_spec,
        any_spec,
        any_spec,
        any_spec,
    ]
    out = pl.pallas_call(
        _copy_body,
        grid=(g,),
        in_specs=specs,
        out_specs=specs,
        out_shape=[
            jax.ShapeDtypeStruct(nodes.shape, nodes.dtype),
            jax.ShapeDtypeStruct(edge_index.shape, edge_index.dtype),
            jax.ShapeDtypeStruct(edges.shape, edges.dtype),
            jax.ShapeDtypeStruct(u.shape, u.dtype),
            jax.ShapeDtypeStruct(batch.shape, batch.dtype),
        ],
        scratch_shapes=[pltpu.SemaphoreType.DMA] * 3,
        input_output_aliases={2: 2},
    )(nodes, edge_index, edges, u, batch)

    return tuple(out)
